# Initial kernel scaffold; baseline (speedup 1.0000x reference)
import jax
import jax.numpy as jnp
from jax.experimental import pallas as pl


def kernel(features, W1, b1, W2, b2):
    raise NotImplementedError("write your pallas kernel here")


# speedup vs baseline: 12.1128x; 12.1128x over previous
import jax
import jax.numpy as jnp
import numpy as np
from jax.experimental import pallas as pl

_TOPK = 31


def _mlp_body(x_ref, w1_ref, b1_ref, w2_ref, b2_ref, emb_ref):
    x = x_ref[...]
    h = jax.lax.dot_general(x, w1_ref[...], (((1,), (1,)), ((), ())),
                            preferred_element_type=jnp.float32)
    h = jnp.maximum(h + b1_ref[...], 0.0)
    h = jax.lax.dot_general(h, w2_ref[...], (((1,), (1,)), ((), ())),
                            preferred_element_type=jnp.float32)
    h = h + b2_ref[...]
    ss = jnp.sum(h * h, axis=1, keepdims=True)
    nrm = jnp.maximum(jnp.sqrt(ss), 1e-12)
    emb_ref[...] = h / nrm


def _sim_body(emb_blk_ref, emb_ref, out_ref):
    s = jax.lax.dot_general(emb_blk_ref[...], emb_ref[...],
                            (((1,), (1,)), ((), ())),
                            preferred_element_type=jnp.float32)
    b = jax.lax.bitcast_convert_type(s, jnp.int32)
    keys = b ^ (jax.lax.shift_right_arithmetic(b, 31) & np.int32(0x7FFFFFFF))
    br = s.shape[0]
    lo0 = jnp.full((br, 1), jnp.iinfo(jnp.int32).min, jnp.int32)
    hi0 = jnp.full((br, 1), jnp.iinfo(jnp.int32).max, jnp.int32)

    def body(_, carry):
        lo, hi = carry
        mid = (lo & hi) + ((lo ^ hi) >> 1)
        cnt = jnp.sum((keys >= mid).astype(jnp.int32), axis=1, keepdims=True)
        take = cnt >= _TOPK
        return jnp.where(take, mid, lo), jnp.where(take, hi, mid)

    lo, _ = jax.lax.fori_loop(0, 32, body, (lo0, hi0))
    out_ref[...] = jnp.where((keys >= lo) & (s > 0.0), s, 0.0)


def kernel(features, W1, b1, W2, b2):
    n, d = features.shape
    emb = pl.pallas_call(
        _mlp_body,
        out_shape=jax.ShapeDtypeStruct((n, d), jnp.float32),
    )(features, W1, b1.reshape(1, d), W2, b2.reshape(1, d))

    br = 200 if n % 200 == 0 else n
    out = pl.pallas_call(
        _sim_body,
        grid=(n // br,),
        in_specs=[
            pl.BlockSpec((br, d), lambda i: (i, 0)),
            pl.BlockSpec((n, d), lambda i: (0, 0)),
        ],
        out_specs=pl.BlockSpec((br, n), lambda i: (i, 0)),
        out_shape=jax.ShapeDtypeStruct((n, n), jnp.float32),
    )(emb, emb)
    return out


# speedup vs baseline: 14.2205x; 1.1740x over previous
import jax
import jax.numpy as jnp
import numpy as np
from jax.experimental import pallas as pl

_TOPK = 31
_KBOUND = int(np.float32(1.00390625).view(np.int32))


def _mlp_body(x_ref, w1_ref, b1_ref, w2_ref, b2_ref, emb_ref):
    x = x_ref[...]
    h = jax.lax.dot_general(x, w1_ref[...], (((1,), (1,)), ((), ())),
                            preferred_element_type=jnp.float32)
    h = jnp.maximum(h + b1_ref[...], 0.0)
    h = jax.lax.dot_general(h, w2_ref[...], (((1,), (1,)), ((), ())),
                            preferred_element_type=jnp.float32)
    h = h + b2_ref[...]
    ss = jnp.sum(h * h, axis=1, keepdims=True)
    nrm = jnp.maximum(jnp.sqrt(ss), 1e-12)
    emb_ref[...] = h / nrm


def _sim_body(emb_blk_ref, emb_ref, out_ref):
    s = jax.lax.dot_general(emb_blk_ref[...], emb_ref[...],
                            (((1,), (1,)), ((), ())),
                            preferred_element_type=jnp.float32)
    b = jax.lax.bitcast_convert_type(s, jnp.int32)
    keys = b ^ (jax.lax.shift_right_arithmetic(b, 31) & np.int32(0x7FFFFFFF))
    br = s.shape[0]

    lo0 = jnp.full((br, 1), np.int32(-_KBOUND), jnp.int32)
    hi0 = jnp.full((br, 1), np.int32(_KBOUND), jnp.int32)
    cnt0 = jnp.full((br, 1), np.int32(s.shape[1]), jnp.int32)
    ukeys = keys.astype(jnp.uint32)

    def cond(carry):
        t, _, _, cnt_lo = carry
        return (t < 31) & jnp.any(cnt_lo != _TOPK)

    def body(carry):
        t, lo, hi, cnt_lo = carry
        mid = (lo + hi) >> 1
        ge = ((mid - 1).astype(jnp.uint32) - ukeys) >> jnp.uint32(31)
        cnt = jnp.sum(ge.astype(jnp.int32), axis=1, keepdims=True)
        take = cnt >= _TOPK
        return (t + 1, jnp.where(take, mid, lo), jnp.where(take, hi, mid),
                jnp.where(take, cnt, cnt_lo))

    _, lo, _, _ = jax.lax.while_loop(
        cond, body, (jnp.int32(0), lo0, hi0, cnt0))
    thr = jnp.maximum(lo, 1)
    out_ref[...] = jnp.where(keys >= thr, s, 0.0)


def kernel(features, W1, b1, W2, b2):
    n, d = features.shape
    emb = pl.pallas_call(
        _mlp_body,
        out_shape=jax.ShapeDtypeStruct((n, d), jnp.float32),
    )(features, W1, b1.reshape(1, d), W2, b2.reshape(1, d))

    br = 200 if n % 200 == 0 else n
    out = pl.pallas_call(
        _sim_body,
        grid=(n // br,),
        in_specs=[
            pl.BlockSpec((br, d), lambda i: (i, 0)),
            pl.BlockSpec((n, d), lambda i: (0, 0)),
        ],
        out_specs=pl.BlockSpec((br, n), lambda i: (i, 0)),
        out_shape=jax.ShapeDtypeStruct((n, n), jnp.float32),
    )(emb, emb)
    return out


# speedup vs baseline: 16.6774x; 1.1728x over previous
import jax
import jax.numpy as jnp
import numpy as np
from jax.experimental import pallas as pl

_TOPK = 31
_KBOUND = int(np.float32(1.00390625).view(np.int32))


def _mlp_body(x_ref, w1_ref, b1_ref, w2_ref, b2_ref, emb_ref):
    x = x_ref[...]
    h = jax.lax.dot_general(x, w1_ref[...], (((1,), (1,)), ((), ())),
                            preferred_element_type=jnp.float32)
    h = jnp.maximum(h + b1_ref[...], 0.0)
    h = jax.lax.dot_general(h, w2_ref[...], (((1,), (1,)), ((), ())),
                            preferred_element_type=jnp.float32)
    h = h + b2_ref[...]
    ss = jnp.sum(h * h, axis=1, keepdims=True)
    nrm = jnp.maximum(jnp.sqrt(ss), 1e-12)
    emb_ref[...] = h / nrm


def _sim_body(emb_blk_ref, emb_ref, out_ref):
    s = jax.lax.dot_general(emb_blk_ref[...], emb_ref[...],
                            (((1,), (1,)), ((), ())),
                            preferred_element_type=jnp.float32)
    b = jax.lax.bitcast_convert_type(s, jnp.int32)
    keys = b ^ (jax.lax.shift_right_arithmetic(b, 31) & np.int32(0x7FFFFFFF))
    br = s.shape[0]

    n = s.shape[1]
    npad = -n % 5120

    def count16(x16, m):
        v = jnp.where(x16 >= m.astype(jnp.int16), jnp.int16(1), jnp.int16(0))
        h = v.shape[1] // 2
        f = v[:, :h] + v[:, h:]
        f = f[:, :h // 2] + f[:, h // 2:]
        f = f[:, :h // 4] + f[:, h // 4:]
        return jnp.sum(f.astype(jnp.int32), axis=1, keepdims=True)

    hp = jnp.pad(jax.lax.shift_right_arithmetic(keys, 16).astype(jnp.int16),
                 ((0, 0), (0, npad)), constant_values=np.int16(-32768))
    lo0 = jnp.full((br, 1), np.int32(-32768), jnp.int32)
    hi0 = jnp.full((br, 1), np.int32(32767), jnp.int32)
    z0 = jnp.zeros((br, 1), jnp.int32)

    def body1(_, carry):
        lo, hi, cnt_hi = carry
        mid = (lo + hi) >> 1
        cnt = count16(hp, mid)
        take = cnt >= _TOPK
        return (jnp.where(take, mid, lo), jnp.where(take, hi, mid),
                jnp.where(take, cnt_hi, cnt))

    t16, _, n_gt = jax.lax.fori_loop(0, 16, body1, (lo0, hi0, z0))
    need = _TOPK - n_gt

    lb = (keys & np.int32(0xFFFF)) - np.int32(32768)
    le32 = jnp.where(jax.lax.shift_right_arithmetic(keys, 16) == t16, lb,
                     np.int32(-32768))
    le = jnp.pad(le32.astype(jnp.int16), ((0, 0), (0, npad)),
                 constant_values=np.int16(-32768))
    c0 = jnp.full((br, 1), np.int32(n), jnp.int32)

    def cond2(carry):
        t, _, _, cnt_lo = carry
        return (t < 16) & jnp.any(cnt_lo != need)

    def body2(carry):
        t, lo, hi, cnt_lo = carry
        mid = (lo + hi) >> 1
        cnt = count16(le, mid)
        take = cnt >= need
        return (t + 1, jnp.where(take, mid, lo), jnp.where(take, hi, mid),
                jnp.where(take, cnt, cnt_lo))

    _, tl, _, _ = jax.lax.while_loop(
        cond2, body2, (jnp.int32(0), lo0, hi0, c0))

    thr = (t16 << 16) + (tl + np.int32(32768))
    thr = jnp.maximum(thr, 1)
    out_ref[...] = jnp.where(keys >= thr, s, 0.0)


def kernel(features, W1, b1, W2, b2):
    n, d = features.shape
    emb = pl.pallas_call(
        _mlp_body,
        out_shape=jax.ShapeDtypeStruct((n, d), jnp.float32),
    )(features, W1, b1.reshape(1, d), W2, b2.reshape(1, d))

    br = 200 if n % 200 == 0 else n
    out = pl.pallas_call(
        _sim_body,
        grid=(n // br,),
        in_specs=[
            pl.BlockSpec((br, d), lambda i: (i, 0)),
            pl.BlockSpec((n, d), lambda i: (0, 0)),
        ],
        out_specs=pl.BlockSpec((br, n), lambda i: (i, 0)),
        out_shape=jax.ShapeDtypeStruct((n, n), jnp.float32),
    )(emb, emb)
    return out


# speedup vs baseline: 17.1162x; 1.0263x over previous
import jax
import jax.numpy as jnp
import numpy as np
from jax.experimental import pallas as pl

_TOPK = 31
_KBOUND = int(np.float32(1.00390625).view(np.int32))


def _mlp_body(x_ref, w1_ref, b1_ref, w2_ref, b2_ref, emb_ref):
    x = x_ref[...]
    h = jax.lax.dot_general(x, w1_ref[...], (((1,), (1,)), ((), ())),
                            preferred_element_type=jnp.float32)
    h = jnp.maximum(h + b1_ref[...], 0.0)
    h = jax.lax.dot_general(h, w2_ref[...], (((1,), (1,)), ((), ())),
                            preferred_element_type=jnp.float32)
    h = h + b2_ref[...]
    ss = jnp.sum(h * h, axis=1, keepdims=True)
    nrm = jnp.maximum(jnp.sqrt(ss), 1e-12)
    emb_ref[...] = h / nrm


def _sim_body(emb_blk_ref, emb_ref, out_ref):
    s = jax.lax.dot_general(emb_blk_ref[...], emb_ref[...],
                            (((1,), (1,)), ((), ())),
                            preferred_element_type=jnp.float32)
    b = jax.lax.bitcast_convert_type(s, jnp.int32)
    keys = b ^ (jax.lax.shift_right_arithmetic(b, 31) & np.int32(0x7FFFFFFF))
    br = s.shape[0]

    n = s.shape[1]
    npad = -n % 2048

    def count16(x16, m):
        v = jnp.where(x16 >= m.astype(jnp.int16), jnp.int16(1), jnp.int16(0))
        h = v.shape[1] // 2
        f = v[:, :h] + v[:, h:]
        f = f[:, :h // 2] + f[:, h // 2:]
        f = f[:, :h // 4] + f[:, h // 4:]
        return jnp.sum(f.astype(jnp.int32), axis=1, keepdims=True)

    hp = jnp.pad(jax.lax.shift_right_arithmetic(keys, 16).astype(jnp.int16),
                 ((0, 0), (0, npad)), constant_values=np.int16(-32768))
    lo0 = jnp.full((br, 1), np.int32(-16257), jnp.int32)
    hi0 = jnp.full((br, 1), np.int32(16384), jnp.int32)
    z0 = jnp.zeros((br, 1), jnp.int32)

    def body1(_, carry):
        lo, hi, cnt_hi = carry
        mid = (lo + hi) >> 1
        cnt = count16(hp, mid)
        take = cnt >= _TOPK
        return (jnp.where(take, mid, lo), jnp.where(take, hi, mid),
                jnp.where(take, cnt_hi, cnt))

    t16, _, n_gt = jax.lax.fori_loop(0, 15, body1, (lo0, hi0, z0))
    need = _TOPK - n_gt

    lb = (keys & np.int32(0xFFFF)) - np.int32(32768)
    le32 = jnp.where(jax.lax.shift_right_arithmetic(keys, 16) == t16, lb,
                     np.int32(-32768))
    le = jnp.pad(le32.astype(jnp.int16), ((0, 0), (0, npad)),
                 constant_values=np.int16(-32768))
    lo2 = jnp.full((br, 1), np.int32(-32768), jnp.int32)
    hi2 = jnp.full((br, 1), np.int32(32768), jnp.int32)
    c0 = jnp.full((br, 1), np.int32(n), jnp.int32)

    def cond2(carry):
        t, _, _, cnt_lo = carry
        return (t < 16) & jnp.any(cnt_lo != need)

    def body2(carry):
        t, lo, hi, cnt_lo = carry
        mid = (lo + hi) >> 1
        cnt = count16(le, mid)
        take = cnt >= need
        return (t + 1, jnp.where(take, mid, lo), jnp.where(take, hi, mid),
                jnp.where(take, cnt, cnt_lo))

    _, tl, _, _ = jax.lax.while_loop(
        cond2, body2, (jnp.int32(0), lo2, hi2, c0))

    thr = (t16 << 16) + (tl + np.int32(32768))
    thr = jnp.maximum(thr, 1)
    out_ref[...] = jnp.where(keys >= thr, s, 0.0)


def kernel(features, W1, b1, W2, b2):
    n, d = features.shape
    emb = pl.pallas_call(
        _mlp_body,
        out_shape=jax.ShapeDtypeStruct((n, d), jnp.float32),
    )(features, W1, b1.reshape(1, d), W2, b2.reshape(1, d))

    br = 200 if n % 200 == 0 else n
    out = pl.pallas_call(
        _sim_body,
        grid=(n // br,),
        in_specs=[
            pl.BlockSpec((br, d), lambda i: (i, 0)),
            pl.BlockSpec((n, d), lambda i: (0, 0)),
        ],
        out_specs=pl.BlockSpec((br, n), lambda i: (i, 0)),
        out_shape=jax.ShapeDtypeStruct((n, n), jnp.float32),
    )(emb, emb)
    return out


# speedup vs baseline: 17.4108x; 1.0172x over previous
import jax
import jax.numpy as jnp
import numpy as np
from jax.experimental import pallas as pl

_TOPK = 31


def _mlp_body(x_ref, w1_ref, b1_ref, w2_ref, b2_ref, emb_ref):
    x = x_ref[...]
    h = jax.lax.dot_general(x, w1_ref[...], (((1,), (1,)), ((), ())),
                            preferred_element_type=jnp.float32)
    h = jnp.maximum(h + b1_ref[...], 0.0)
    h = jax.lax.dot_general(h, w2_ref[...], (((1,), (1,)), ((), ())),
                            preferred_element_type=jnp.float32)
    h = h + b2_ref[...]
    ss = jnp.sum(h * h, axis=1, keepdims=True)
    nrm = jnp.maximum(jnp.sqrt(ss), 1e-12)
    emb_ref[...] = h / nrm


def _sim_body(emb_blk_ref, emb_ref, out_ref):
    s = jax.lax.dot_general(emb_blk_ref[...], emb_ref[...],
                            (((1,), (1,)), ((), ())),
                            preferred_element_type=jnp.float32)
    b = jax.lax.bitcast_convert_type(s, jnp.int32)
    keys = b ^ (jax.lax.shift_right_arithmetic(b, 31) & np.int32(0x7FFFFFFF))
    br = s.shape[0]

    n = s.shape[1]
    npad = -n % 2048

    def count16(x16, m):
        v = jnp.where(x16 >= m.astype(jnp.int16), jnp.int16(1), jnp.int16(0))
        h = v.shape[1] // 2
        f = v[:, :h] + v[:, h:]
        f = f[:, :h // 2] + f[:, h // 2:]
        f = f[:, :h // 4] + f[:, h // 4:]
        return jnp.sum(f.astype(jnp.int32), axis=1, keepdims=True)

    z0 = jnp.zeros((br, 1), jnp.int32)

    hp = jnp.pad(jax.lax.shift_right_arithmetic(keys, 16).astype(jnp.int16),
                 ((0, 0), (0, npad)), constant_values=np.int16(-32768))
    lo0 = jnp.full((br, 1), np.int32(-16257), jnp.int32)
    hi0 = jnp.full((br, 1), np.int32(16384), jnp.int32)

    def body1(_, carry):
        lo, hi, cnt_hi = carry
        mid = (lo + hi) >> 1
        cnt = count16(hp, mid)
        take = cnt >= _TOPK
        return (jnp.where(take, mid, lo), jnp.where(take, hi, mid),
                jnp.where(take, cnt_hi, cnt))

    t16, _, n_gt = jax.lax.fori_loop(0, 15, body1, (lo0, hi0, z0))
    need = _TOPK - n_gt

    lb = (keys & np.int32(0xFFFF)) - np.int32(32768)
    le32 = jnp.where(jax.lax.shift_right_arithmetic(keys, 16) == t16, lb,
                     np.int32(-32768))
    le = jnp.pad(le32.astype(jnp.int16), ((0, 0), (0, npad)),
                 constant_values=np.int16(-32768))
    nt = count16(le, jnp.full((br, 1), np.int32(-32767), jnp.int32))
    loC = jnp.where(nt >= need, np.int32(-32767), np.int32(-32768))
    hiC = jnp.full((br, 1), np.int32(32768), jnp.int32)

    def cond2(carry):
        t, _, _, cnt_lo = carry
        return (t < 16) & jnp.any(cnt_lo != need)

    def body2(carry):
        t, lo, hi, cnt_lo = carry
        mid = (lo + hi) >> 1
        cnt = count16(le, mid)
        take = cnt >= need
        return (t + 1, jnp.where(take, mid, lo), jnp.where(take, hi, mid),
                jnp.where(take, cnt, cnt_lo))

    _, tl, _, _ = jax.lax.while_loop(
        cond2, body2, (jnp.int32(0), loC, hiC, nt))

    thr = (t16 << 16) + (tl + np.int32(32768))
    thr = jnp.maximum(thr, 1)
    out_ref[...] = jnp.where(keys >= thr, s, 0.0)


def kernel(features, W1, b1, W2, b2):
    n, d = features.shape
    emb = pl.pallas_call(
        _mlp_body,
        out_shape=jax.ShapeDtypeStruct((n, d), jnp.float32),
    )(features, W1, b1.reshape(1, d), W2, b2.reshape(1, d))

    br = 200 if n % 200 == 0 else n
    out = pl.pallas_call(
        _sim_body,
        grid=(n // br,),
        in_specs=[
            pl.BlockSpec((br, d), lambda i: (i, 0)),
            pl.BlockSpec((n, d), lambda i: (0, 0)),
        ],
        out_specs=pl.BlockSpec((br, n), lambda i: (i, 0)),
        out_shape=jax.ShapeDtypeStruct((n, n), jnp.float32),
    )(emb, emb)
    return out


# speedup vs baseline: 19.1069x; 1.0974x over previous
import jax
import jax.numpy as jnp
import numpy as np
from jax.experimental import pallas as pl

_TOPK = 31


def _mlp_body(x_ref, w1_ref, b1_ref, w2_ref, b2_ref, emb_ref):
    x = x_ref[...]
    h = jax.lax.dot_general(x, w1_ref[...], (((1,), (1,)), ((), ())),
                            preferred_element_type=jnp.float32)
    h = jnp.maximum(h + b1_ref[...], 0.0)
    h = jax.lax.dot_general(h, w2_ref[...], (((1,), (1,)), ((), ())),
                            preferred_element_type=jnp.float32)
    h = h + b2_ref[...]
    ss = jnp.sum(h * h, axis=1, keepdims=True)
    nrm = jnp.maximum(jnp.sqrt(ss), 1e-12)
    emb_ref[...] = h / nrm


def _sim_body(emb_blk_ref, emb_ref, out_ref):
    s = jax.lax.dot_general(emb_blk_ref[...], emb_ref[...],
                            (((1,), (1,)), ((), ())),
                            preferred_element_type=jnp.float32)
    b = jax.lax.bitcast_convert_type(s, jnp.int32)
    keys = b ^ (jax.lax.shift_right_arithmetic(b, 31) & np.int32(0x7FFFFFFF))
    br = s.shape[0]

    n = s.shape[1]
    npad = -n % 2048

    def count16(x16, m):
        v = jnp.where(x16 >= m.astype(jnp.int16), jnp.int16(1), jnp.int16(0))
        h = v.shape[1] // 2
        f = v[:, :h] + v[:, h:]
        f = f[:, :h // 2] + f[:, h // 2:]
        f = f[:, :h // 4] + f[:, h // 4:]
        f = f[:, :h // 8] + f[:, h // 8:]
        return jnp.sum(f.astype(jnp.int32), axis=1, keepdims=True)

    z0 = jnp.zeros((br, 1), jnp.int32)

    hp = jnp.pad(jax.lax.shift_right_arithmetic(keys, 16).astype(jnp.int16),
                 ((0, 0), (0, npad)), constant_values=np.int16(-32768))
    lo0 = jnp.full((br, 1), np.int32(-1), jnp.int32)
    hi0 = jnp.full((br, 1), np.int32(16257), jnp.int32)

    def body1(_, carry):
        lo, hi, cnt_hi = carry
        mid = (lo + hi) >> 1
        cnt = count16(hp, mid)
        take = cnt >= _TOPK
        return (jnp.where(take, mid, lo), jnp.where(take, hi, mid),
                jnp.where(take, cnt_hi, cnt))

    t16, _, n_gt = jax.lax.fori_loop(0, 14, body1, (lo0, hi0, z0))
    need = _TOPK - n_gt

    lb = (keys & np.int32(0xFFFF)) - np.int32(32768)
    le32 = jnp.where(jax.lax.shift_right_arithmetic(keys, 16) == t16, lb,
                     np.int32(-32768))
    le = jnp.pad(le32.astype(jnp.int16), ((0, 0), (0, npad)),
                 constant_values=np.int16(-32768))
    nt = count16(le, jnp.full((br, 1), np.int32(-32767), jnp.int32))
    loC = jnp.where(nt >= need, np.int32(-32767), np.int32(-32768))
    hiC = jnp.full((br, 1), np.int32(32768), jnp.int32)

    def cond2(carry):
        t, _, _, cnt_lo = carry
        return (t < 16) & jnp.any(cnt_lo != need)

    def body2(carry):
        t, lo, hi, cnt_lo = carry
        mid = (lo + hi) >> 1
        cnt = count16(le, mid)
        take = cnt >= need
        return (t + 1, jnp.where(take, mid, lo), jnp.where(take, hi, mid),
                jnp.where(take, cnt, cnt_lo))

    _, tl, _, _ = jax.lax.while_loop(
        cond2, body2, (jnp.int32(0), loC, hiC, nt))

    thr = (t16 << 16) + (tl + np.int32(32768))
    thr = jnp.maximum(thr, 1)
    out_ref[...] = jnp.where(keys >= thr, s, 0.0)


def kernel(features, W1, b1, W2, b2):
    n, d = features.shape
    emb = pl.pallas_call(
        _mlp_body,
        out_shape=jax.ShapeDtypeStruct((n, d), jnp.float32),
    )(features, W1, b1.reshape(1, d), W2, b2.reshape(1, d))

    br = 200 if n % 200 == 0 else n
    out = pl.pallas_call(
        _sim_body,
        grid=(n // br,),
        in_specs=[
            pl.BlockSpec((br, d), lambda i: (i, 0)),
            pl.BlockSpec((n, d), lambda i: (0, 0)),
        ],
        out_specs=pl.BlockSpec((br, n), lambda i: (i, 0)),
        out_shape=jax.ShapeDtypeStruct((n, n), jnp.float32),
    )(emb, emb)
    return out
